# baseline (speedup 1.0000x reference)
import functools

import jax
import jax.numpy as jnp
from jax import lax
from jax.experimental import pallas as pl
from jax.experimental.pallas import tpu as pltpu
from jax.experimental.pallas import tpu_sc as plsc

B = 16384
F = 26
V = 1000
D = 128
L = 16

NC = 2
NS = 16
NW = NC * NS
TPW = B // NW
T = 128
NSUB = TPW // T
KD = D // L

_mesh = plsc.VectorSubcoreMesh(core_axis_name="c", subcore_axis_name="s")

_GATHER_DNUMS = lax.GatherDimensionNumbers(
    offset_dims=(), collapsed_slice_dims=(0,), start_index_map=(0,))


def _hsum(v, lanes):
    for sh in (8, 4, 2, 1):
        idx = lax.bitwise_xor(lanes, sh).reshape(L, 1)
        v = v + lax.gather(v, idx, _GATHER_DNUMS, slice_sizes=(1,),
                           mode=lax.GatherScatterMode.PROMISE_IN_BOUNDS)
    return v


@functools.partial(
    pl.kernel,
    out_type=jax.ShapeDtypeStruct((B, D), jnp.float32),
    mesh=_mesh,
    scratch_types=[
        pltpu.VMEM((F, T), jnp.int32),
        pltpu.VMEM((F, T), jnp.int32),
        pltpu.VMEM((T, D), jnp.float32),
        pltpu.VMEM((T, D), jnp.float32),
        pltpu.VMEM((T, D), jnp.float32),
        pltpu.VMEM((T, D), jnp.float32),
        pltpu.VMEM((D,), jnp.float32),
        pltpu.VMEM((D,), jnp.float32),
        pltpu.SemaphoreType.DMA,
        pltpu.SemaphoreType.DMA,
        pltpu.SemaphoreType.DMA,
        pltpu.SemaphoreType.DMA,
    ],
)
def _emb_ln_kernel(tokT_hbm, tab_hbm, scale_hbm, bias_hbm, out_hbm,
                   idx0, idx1, acc0, acc1, ob0, ob1,
                   scale_v, bias_v, g0, g1, o0, o1):
    wid = lax.axis_index("s") * NC + lax.axis_index("c")
    base = wid * TPW

    pltpu.sync_copy(scale_hbm, scale_v)
    pltpu.sync_copy(bias_hbm, bias_v)

    idx_b = (idx0, idx1)
    acc_b = (acc0, acc1)
    ob_b = (ob0, ob1)
    g_b = (g0, g1)
    o_b = (o0, o1)

    def stage(j):
        p = j % 2
        idx_v, acc, gsem = idx_b[p], acc_b[p], g_b[p]
        pltpu.sync_copy(tokT_hbm.at[wid * NSUB + j], idx_v)

        @pl.loop(0, T // L)
        def _idx(tb):
            for f in range(F):
                v = idx_v[f, pl.ds(tb * L, L)]
                idx_v[f, pl.ds(tb * L, L)] = (
                    jnp.minimum(jnp.maximum(v, 0), V - 1) + f * V)

        zeros = jnp.zeros((L,), jnp.float32)

        @pl.loop(0, T)
        def _zero(r):
            for k in range(KD):
                acc[r, pl.ds(k * L, L)] = zeros

        return [[pltpu.async_copy(tab_hbm.at[idx_v.at[f]], acc, gsem,
                                  add=True)
                 for f in range(F)]]

    def finish_range(j, lo, n):
        p = j % 2
        acc, ob, osem = acc_b[p], ob_b[p], o_b[p]
        t0 = base + j * T
        lanes = lax.iota(jnp.int32, L)

        @pl.loop(lo, lo + n)
        def _tok(t):
            accs = [acc[t, pl.ds(k * L, L)] for k in range(KD)]
            s = accs[0]
            for k in range(1, KD):
                s = s + accs[k]
            mean = _hsum(s, lanes) * jnp.float32(1.0 / D)
            dif = [a - mean for a in accs]
            vv = dif[0] * dif[0]
            for k in range(1, KD):
                vv = vv + dif[k] * dif[k]
            x = _hsum(vv, lanes) * jnp.float32(1.0 / D) + jnp.float32(1e-12)
            i = lax.bitcast_convert_type(x, jnp.int32)
            i = jnp.int32(0x5F3759DF) - lax.shift_right_logical(i, 1)
            y = lax.bitcast_convert_type(i, jnp.float32)
            half = x * jnp.float32(0.5)
            for _ in range(4):
                y = y * (jnp.float32(1.5) - half * y * y)
            for k in range(KD):
                o = (dif[k] * y * scale_v[pl.ds(k * L, L)]
                     + bias_v[pl.ds(k * L, L)])
                ob[t, pl.ds(k * L, L)] = o

        return pltpu.async_copy(ob.at[pl.ds(lo, n)],
                                out_hbm.at[pl.ds(t0 + lo, n)], osem)

    out_pending = {0: [], 1: []}
    gather_h = {0: stage(0)}
    for j in range(NSUB):
        if j + 1 < NSUB:
            gather_h[j + 1] = stage(j + 1)
        p = j % 2
        waves = gather_h.pop(j)
        tw = T // len(waves)
        for w, wave in enumerate(waves):
            for c in wave:
                c.wait()
            if w == 0:
                for h in out_pending[p]:
                    h.wait()
                out_pending[p] = []
            out_pending[p].append(finish_range(j, w * tw, tw))
    for p in (0, 1):
        for h in out_pending[p]:
            h.wait()


def kernel(tokens, eval, tables, ln_scale, ln_bias):
    tok_c = tokens.astype(jnp.int32).reshape(B // T, T, F).transpose(0, 2, 1)
    tab_flat = tables.reshape(F * V, D)
    return _emb_ln_kernel(tok_c, tab_flat, ln_scale, ln_bias)


# speedup vs baseline: 1.0063x; 1.0063x over previous
import functools

import jax
import jax.numpy as jnp
from jax import lax
from jax.experimental import pallas as pl
from jax.experimental.pallas import tpu as pltpu
from jax.experimental.pallas import tpu_sc as plsc

B = 16384
F = 26
V = 1000
D = 128
L = 16

NC = 2
NS = 16
NW = NC * NS
TPW = B // NW
T = 128
NSUB = TPW // T
KD = D // L

_mesh = plsc.VectorSubcoreMesh(core_axis_name="c", subcore_axis_name="s")

_GATHER_DNUMS = lax.GatherDimensionNumbers(
    offset_dims=(), collapsed_slice_dims=(0,), start_index_map=(0,))


def _hsum(v, lanes):
    for sh in (8, 4, 2, 1):
        idx = lax.bitwise_xor(lanes, sh).reshape(L, 1)
        v = v + lax.gather(v, idx, _GATHER_DNUMS, slice_sizes=(1,),
                           mode=lax.GatherScatterMode.PROMISE_IN_BOUNDS)
    return v


@functools.partial(
    pl.kernel,
    out_type=jax.ShapeDtypeStruct((B, D), jnp.float32),
    mesh=_mesh,
    scratch_types=[
        pltpu.VMEM((F, T), jnp.int32),
        pltpu.VMEM((F, T), jnp.int32),
        pltpu.VMEM((T, D), jnp.float32),
        pltpu.VMEM((T, D), jnp.float32),
        pltpu.VMEM((T, D), jnp.float32),
        pltpu.VMEM((T, D), jnp.float32),
        pltpu.VMEM((D,), jnp.float32),
        pltpu.VMEM((D,), jnp.float32),
        pltpu.SemaphoreType.DMA,
        pltpu.SemaphoreType.DMA,
        pltpu.SemaphoreType.DMA,
        pltpu.SemaphoreType.DMA,
    ],
)
def _emb_ln_kernel(tokT_hbm, tab_hbm, scale_hbm, bias_hbm, out_hbm,
                   idx0, idx1, acc0, acc1, ob0, ob1,
                   scale_v, bias_v, g0, g1, o0, o1):
    wid = lax.axis_index("s") * NC + lax.axis_index("c")
    base = wid * TPW

    pltpu.sync_copy(scale_hbm, scale_v)
    pltpu.sync_copy(bias_hbm, bias_v)

    idx_b = (idx0, idx1)
    acc_b = (acc0, acc1)
    ob_b = (ob0, ob1)
    g_b = (g0, g1)
    o_b = (o0, o1)

    def stage(j):
        p = j % 2
        idx_v, acc, gsem = idx_b[p], acc_b[p], g_b[p]
        t0 = base + j * T
        pltpu.sync_copy(tokT_hbm.at[:, pl.ds(t0, T)], idx_v)

        @pl.loop(0, T // L)
        def _idx(tb):
            for f in range(F):
                v = idx_v[f, pl.ds(tb * L, L)]
                idx_v[f, pl.ds(tb * L, L)] = (
                    jnp.minimum(jnp.maximum(v, 0), V - 1) + f * V)

        zeros = jnp.zeros((L,), jnp.float32)

        @pl.loop(0, T)
        def _zero(r):
            for k in range(KD):
                acc[r, pl.ds(k * L, L)] = zeros

        return [[pltpu.async_copy(tab_hbm.at[idx_v.at[f]], acc, gsem,
                                  add=True)
                 for f in range(F)]]

    def finish_range(j, lo, n):
        p = j % 2
        acc, ob, osem = acc_b[p], ob_b[p], o_b[p]
        t0 = base + j * T
        lanes = lax.iota(jnp.int32, L)

        @pl.loop(lo, lo + n)
        def _tok(t):
            accs = [acc[t, pl.ds(k * L, L)] for k in range(KD)]
            s = accs[0]
            for k in range(1, KD):
                s = s + accs[k]
            mean = _hsum(s, lanes) * jnp.float32(1.0 / D)
            dif = [a - mean for a in accs]
            vv = dif[0] * dif[0]
            for k in range(1, KD):
                vv = vv + dif[k] * dif[k]
            x = _hsum(vv, lanes) * jnp.float32(1.0 / D) + jnp.float32(1e-12)
            i = lax.bitcast_convert_type(x, jnp.int32)
            i = jnp.int32(0x5F3759DF) - lax.shift_right_logical(i, 1)
            y = lax.bitcast_convert_type(i, jnp.float32)
            half = x * jnp.float32(0.5)
            for _ in range(4):
                y = y * (jnp.float32(1.5) - half * y * y)
            for k in range(KD):
                o = (dif[k] * y * scale_v[pl.ds(k * L, L)]
                     + bias_v[pl.ds(k * L, L)])
                ob[t, pl.ds(k * L, L)] = o

        return pltpu.async_copy(ob.at[pl.ds(lo, n)],
                                out_hbm.at[pl.ds(t0 + lo, n)], osem)

    out_pending = {0: [], 1: []}
    gather_h = {0: stage(0)}
    for j in range(NSUB):
        if j + 1 < NSUB:
            gather_h[j + 1] = stage(j + 1)
        p = j % 2
        waves = gather_h.pop(j)
        tw = T // len(waves)
        for w, wave in enumerate(waves):
            for c in wave:
                c.wait()
            if w == 0:
                for h in out_pending[p]:
                    h.wait()
                out_pending[p] = []
            out_pending[p].append(finish_range(j, w * tw, tw))
    for p in (0, 1):
        for h in out_pending[p]:
            h.wait()


def kernel(tokens, eval, tables, ln_scale, ln_bias):
    tok_t = tokens.astype(jnp.int32).T
    tab_flat = tables.reshape(F * V, D)
    return _emb_ln_kernel(tok_t, tab_flat, ln_scale, ln_bias)


# speedup vs baseline: 1.0152x; 1.0088x over previous
import functools

import jax
import jax.numpy as jnp
from jax import lax
from jax.experimental import pallas as pl
from jax.experimental.pallas import tpu as pltpu
from jax.experimental.pallas import tpu_sc as plsc

B = 16384
F = 26
V = 1000
D = 128
L = 16

NC = 2
NS = 16
NW = NC * NS
TPW = B // NW
T = 128
NSUB = TPW // T
KD = D // L

_mesh = plsc.VectorSubcoreMesh(core_axis_name="c", subcore_axis_name="s")

_GATHER_DNUMS = lax.GatherDimensionNumbers(
    offset_dims=(), collapsed_slice_dims=(0,), start_index_map=(0,))


def _hsum(v, lanes):
    for sh in (8, 4, 2, 1):
        idx = lax.bitwise_xor(lanes, sh).reshape(L, 1)
        v = v + lax.gather(v, idx, _GATHER_DNUMS, slice_sizes=(1,),
                           mode=lax.GatherScatterMode.PROMISE_IN_BOUNDS)
    return v


@functools.partial(
    pl.kernel,
    out_type=jax.ShapeDtypeStruct((B, D), jnp.float32),
    mesh=_mesh,
    scratch_types=[
        pltpu.VMEM((F, T), jnp.int32),
        pltpu.VMEM((F, T), jnp.int32),
        pltpu.VMEM((F, T), jnp.int32),
        pltpu.VMEM((T, D), jnp.float32),
        pltpu.VMEM((T, D), jnp.float32),
        pltpu.VMEM((T, D), jnp.float32),
        pltpu.VMEM((T, D), jnp.float32),
        pltpu.VMEM((T, D), jnp.float32),
        pltpu.VMEM((T, D), jnp.float32),
        pltpu.VMEM((D,), jnp.float32),
        pltpu.VMEM((D,), jnp.float32),
        pltpu.SemaphoreType.DMA,
        pltpu.SemaphoreType.DMA,
        pltpu.SemaphoreType.DMA,
        pltpu.SemaphoreType.DMA,
        pltpu.SemaphoreType.DMA,
        pltpu.SemaphoreType.DMA,
    ],
)
def _emb_ln_kernel(tokT_hbm, tab_hbm, scale_hbm, bias_hbm, out_hbm,
                   idx0, idx1, idx2, acc0, acc1, acc2, ob0, ob1, ob2,
                   scale_v, bias_v, g0, g1, g2, o0, o1, o2):
    wid = lax.axis_index("s") * NC + lax.axis_index("c")
    base = wid * TPW

    pltpu.sync_copy(scale_hbm, scale_v)
    pltpu.sync_copy(bias_hbm, bias_v)

    idx_b = (idx0, idx1, idx2)
    acc_b = (acc0, acc1, acc2)
    ob_b = (ob0, ob1, ob2)
    g_b = (g0, g1, g2)
    o_b = (o0, o1, o2)

    def stage(j):
        p = j % 3
        idx_v, acc, gsem = idx_b[p], acc_b[p], g_b[p]
        t0 = base + j * T
        pltpu.sync_copy(tokT_hbm.at[:, pl.ds(t0, T)], idx_v)

        @pl.loop(0, T // L)
        def _idx(tb):
            for f in range(F):
                v = idx_v[f, pl.ds(tb * L, L)]
                idx_v[f, pl.ds(tb * L, L)] = (
                    jnp.minimum(jnp.maximum(v, 0), V - 1) + f * V)

        zeros = jnp.zeros((L,), jnp.float32)

        @pl.loop(0, T)
        def _zero(r):
            for k in range(KD):
                acc[r, pl.ds(k * L, L)] = zeros

        return [[pltpu.async_copy(tab_hbm.at[idx_v.at[f]], acc, gsem,
                                  add=True)
                 for f in range(F)]]

    def finish_range(j, lo, n):
        p = j % 3
        acc, ob, osem = acc_b[p], ob_b[p], o_b[p]
        t0 = base + j * T
        lanes = lax.iota(jnp.int32, L)

        @pl.loop(lo, lo + n)
        def _tok(t):
            accs = [acc[t, pl.ds(k * L, L)] for k in range(KD)]
            s = accs[0]
            for k in range(1, KD):
                s = s + accs[k]
            mean = _hsum(s, lanes) * jnp.float32(1.0 / D)
            dif = [a - mean for a in accs]
            vv = dif[0] * dif[0]
            for k in range(1, KD):
                vv = vv + dif[k] * dif[k]
            x = _hsum(vv, lanes) * jnp.float32(1.0 / D) + jnp.float32(1e-12)
            i = lax.bitcast_convert_type(x, jnp.int32)
            i = jnp.int32(0x5F3759DF) - lax.shift_right_logical(i, 1)
            y = lax.bitcast_convert_type(i, jnp.float32)
            half = x * jnp.float32(0.5)
            for _ in range(4):
                y = y * (jnp.float32(1.5) - half * y * y)
            for k in range(KD):
                o = (dif[k] * y * scale_v[pl.ds(k * L, L)]
                     + bias_v[pl.ds(k * L, L)])
                ob[t, pl.ds(k * L, L)] = o

        return pltpu.async_copy(ob.at[pl.ds(lo, n)],
                                out_hbm.at[pl.ds(t0 + lo, n)], osem)

    out_pending = {0: [], 1: [], 2: []}
    gather_h = {0: stage(0)}
    if NSUB > 1:
        gather_h[1] = stage(1)
    for j in range(NSUB):
        if j + 2 < NSUB:
            gather_h[j + 2] = stage(j + 2)
        p = j % 3
        for wave in gather_h.pop(j):
            for c in wave:
                c.wait()
        for h in out_pending[p]:
            h.wait()
        out_pending[p] = [finish_range(j, 0, T)]
    for p in (0, 1, 2):
        for h in out_pending[p]:
            h.wait()


def kernel(tokens, eval, tables, ln_scale, ln_bias):
    tok_t = tokens.astype(jnp.int32).T
    tab_flat = tables.reshape(F * V, D)
    return _emb_ln_kernel(tok_t, tab_flat, ln_scale, ln_bias)
